# Initial kernel scaffold; baseline (speedup 1.0000x reference)
import jax
import jax.numpy as jnp
from jax.experimental import pallas as pl


def kernel(LUT, x):
    raise NotImplementedError("write your pallas kernel here")


# speedup vs baseline: 179.0226x; 179.0226x over previous
import functools

import jax
import jax.numpy as jnp
from jax import lax
from jax.experimental import pallas as pl
from jax.experimental.pallas import tpu as pltpu
from jax.experimental.pallas import tpu_sc as plsc

DIM = 33
NLUT = 3 * DIM ** 3
NC, NS, L = 2, 16, 16
NW = NC * NS
CHUNK = 2048


def kernel(LUT, x):
    B, C, W, H = x.shape
    P = W * H
    N = B * P
    per_w = N // NW
    n_chunks = per_w // CHUNK

    x_flat = x.reshape(B * C, P)
    lut_flat = LUT.reshape(NLUT)
    binsize = jnp.float32(1.000001 / (DIM - 1))

    mesh = plsc.VectorSubcoreMesh(
        core_axis_name="c", subcore_axis_name="s", num_cores=NC, num_subcores=NS
    )

    @functools.partial(
        pl.kernel,
        out_type=jax.ShapeDtypeStruct((B * C, P), jnp.float32),
        mesh=mesh,
        compiler_params=pltpu.CompilerParams(needs_layout_passes=False),
        scratch_types=[
            pltpu.VMEM((NLUT,), jnp.float32),
            pltpu.VMEM((CHUNK,), jnp.float32),
            pltpu.VMEM((CHUNK,), jnp.float32),
            pltpu.VMEM((CHUNK,), jnp.float32),
            pltpu.VMEM((CHUNK,), jnp.float32),
            pltpu.VMEM((CHUNK,), jnp.float32),
            pltpu.VMEM((CHUNK,), jnp.float32),
        ],
    )
    def lut_kernel(lut_hbm, x_hbm, out_hbm, lut_v, r_v, g_v, b_v, or_v, og_v, ob_v):
        in_v = (r_v, g_v, b_v)
        out_v = (or_v, og_v, ob_v)
        wid = lax.axis_index("s") * NC + lax.axis_index("c")
        batch = wid // 4
        base_px = (wid % 4) * per_w
        row0 = 3 * batch

        pltpu.sync_copy(lut_hbm, lut_v)

        def chunk_body(j, _):
            start = base_px + j * CHUNK
            for c in range(3):
                pltpu.sync_copy(x_hbm.at[row0 + c, pl.ds(start, CHUNK)], in_v[c])

            def px_body(i, _):
                off = i * L
                r = in_v[0][pl.ds(off, L)]
                g = in_v[1][pl.ds(off, L)]
                b = in_v[2][pl.ds(off, L)]
                rs = r / binsize
                gs = g / binsize
                bs = b / binsize
                rid = jnp.clip(rs.astype(jnp.int32), 0, DIM - 2)
                gid = jnp.clip(gs.astype(jnp.int32), 0, DIM - 2)
                bid = jnp.clip(bs.astype(jnp.int32), 0, DIM - 2)
                rd = rs - rid.astype(jnp.float32)
                gd = gs - gid.astype(jnp.float32)
                bd = bs - bid.astype(jnp.float32)
                base = rid + gid * DIM + bid * (DIM * DIM)

                ar = 1.0 - rd
                ag = 1.0 - gd
                ab = 1.0 - bd
                p00 = ag * ab
                p10 = gd * ab
                p01 = ag * bd
                p11 = gd * bd
                w = (ar * p00, rd * p00, ar * p10, rd * p10,
                     ar * p01, rd * p01, ar * p11, rd * p11)
                offs = (0, 1, DIM, DIM + 1,
                        DIM * DIM, DIM * DIM + 1, DIM * DIM + DIM, DIM * DIM + DIM + 1)
                for c in range(3):
                    basec = base + c * (DIM ** 3)
                    acc = w[0] * plsc.load_gather(lut_v, [basec])
                    for k in range(1, 8):
                        acc = acc + w[k] * plsc.load_gather(lut_v, [basec + offs[k]])
                    out_v[c][pl.ds(off, L)] = acc
                return 0

            lax.fori_loop(0, CHUNK // L, px_body, 0)

            for c in range(3):
                pltpu.sync_copy(out_v[c], out_hbm.at[row0 + c, pl.ds(start, CHUNK)])
            return 0

        lax.fori_loop(0, n_chunks, chunk_body, 0)

    out = lut_kernel(lut_flat, x_flat)
    return out.reshape(B, C, W, H)


# speedup vs baseline: 179.3266x; 1.0017x over previous
import functools

import jax
import jax.numpy as jnp
from jax import lax
from jax.experimental import pallas as pl
from jax.experimental.pallas import tpu as pltpu
from jax.experimental.pallas import tpu_sc as plsc

DIM = 33
NLUT = 3 * DIM ** 3
NC, NS, L = 2, 16, 16
NW = NC * NS
CHUNK = 2048


def kernel(LUT, x):
    B, C, W, H = x.shape
    P = W * H
    N = B * P
    per_w = N // NW
    n_chunks = per_w // CHUNK

    x_flat = x.reshape(B * C, P)
    lut_flat = LUT.reshape(NLUT)
    inv_binsize = jnp.float32((DIM - 1) / 1.000001)

    mesh = plsc.VectorSubcoreMesh(
        core_axis_name="c", subcore_axis_name="s", num_cores=NC, num_subcores=NS
    )

    @functools.partial(
        pl.kernel,
        out_type=jax.ShapeDtypeStruct((B * C, P), jnp.float32),
        mesh=mesh,
        compiler_params=pltpu.CompilerParams(needs_layout_passes=False),
        scratch_types=[
            pltpu.VMEM((NLUT,), jnp.float32),
            pltpu.VMEM((CHUNK,), jnp.float32),
            pltpu.VMEM((CHUNK,), jnp.float32),
            pltpu.VMEM((CHUNK,), jnp.float32),
            pltpu.VMEM((CHUNK,), jnp.float32),
            pltpu.VMEM((CHUNK,), jnp.float32),
            pltpu.VMEM((CHUNK,), jnp.float32),
        ],
    )
    def lut_kernel(lut_hbm, x_hbm, out_hbm, lut_v, r_v, g_v, b_v, or_v, og_v, ob_v):
        in_v = (r_v, g_v, b_v)
        out_v = (or_v, og_v, ob_v)
        wid = lax.axis_index("s") * NC + lax.axis_index("c")
        batch = wid // 4
        base_px = (wid % 4) * per_w
        row0 = 3 * batch

        pltpu.sync_copy(lut_hbm, lut_v)

        def chunk_body(j, _):
            start = base_px + j * CHUNK
            for c in range(3):
                pltpu.sync_copy(x_hbm.at[row0 + c, pl.ds(start, CHUNK)], in_v[c])

            def px_body(i, _):
                off = i * L
                r = in_v[0][pl.ds(off, L)]
                g = in_v[1][pl.ds(off, L)]
                b = in_v[2][pl.ds(off, L)]
                rs = r * inv_binsize
                gs = g * inv_binsize
                bs = b * inv_binsize
                rid = jnp.clip(rs.astype(jnp.int32), 0, DIM - 2)
                gid = jnp.clip(gs.astype(jnp.int32), 0, DIM - 2)
                bid = jnp.clip(bs.astype(jnp.int32), 0, DIM - 2)
                rd = rs - rid.astype(jnp.float32)
                gd = gs - gid.astype(jnp.float32)
                bd = bs - bid.astype(jnp.float32)
                base = rid + gid * DIM + bid * (DIM * DIM)

                ar = 1.0 - rd
                ag = 1.0 - gd
                ab = 1.0 - bd
                p00 = ag * ab
                p10 = gd * ab
                p01 = ag * bd
                p11 = gd * bd
                w = (ar * p00, rd * p00, ar * p10, rd * p10,
                     ar * p01, rd * p01, ar * p11, rd * p11)
                offs = (0, 1, DIM, DIM + 1,
                        DIM * DIM, DIM * DIM + 1, DIM * DIM + DIM, DIM * DIM + DIM + 1)
                for c in range(3):
                    basec = base + c * (DIM ** 3)
                    acc = w[0] * plsc.load_gather(lut_v, [basec])
                    for k in range(1, 8):
                        acc = acc + w[k] * plsc.load_gather(lut_v, [basec + offs[k]])
                    out_v[c][pl.ds(off, L)] = acc
                return 0

            lax.fori_loop(0, CHUNK // L, px_body, 0)

            for c in range(3):
                pltpu.sync_copy(out_v[c], out_hbm.at[row0 + c, pl.ds(start, CHUNK)])
            return 0

        lax.fori_loop(0, n_chunks, chunk_body, 0)

    out = lut_kernel(lut_flat, x_flat)
    return out.reshape(B, C, W, H)


# speedup vs baseline: 210.0286x; 1.1712x over previous
import functools

import jax
import jax.numpy as jnp
from jax import lax
from jax.experimental import pallas as pl
from jax.experimental.pallas import tpu as pltpu
from jax.experimental.pallas import tpu_sc as plsc

DIM = 33
NLUT = 3 * DIM ** 3
NC, NS, L = 2, 16, 16
NW = NC * NS
CHUNK = 2048


def kernel(LUT, x):
    B, C, W, H = x.shape
    P = W * H
    N = B * P
    per_w = N // NW
    n_chunks = per_w // CHUNK

    x_flat = x.reshape(B * C, P)
    lut_flat = LUT.reshape(NLUT)
    inv_binsize = jnp.float32((DIM - 1) / 1.000001)

    mesh = plsc.VectorSubcoreMesh(
        core_axis_name="c", subcore_axis_name="s", num_cores=NC, num_subcores=NS
    )

    @functools.partial(
        pl.kernel,
        out_type=jax.ShapeDtypeStruct((B * C, P), jnp.float32),
        mesh=mesh,
        compiler_params=pltpu.CompilerParams(needs_layout_passes=False),
        scratch_types=[
            pltpu.VMEM((NLUT,), jnp.float32),
            pltpu.VMEM((CHUNK,), jnp.float32),
            pltpu.VMEM((CHUNK,), jnp.float32),
            pltpu.VMEM((CHUNK,), jnp.float32),
            pltpu.VMEM((CHUNK,), jnp.float32),
            pltpu.VMEM((CHUNK,), jnp.float32),
            pltpu.VMEM((CHUNK,), jnp.float32),
        ],
    )
    def lut_kernel(lut_hbm, x_hbm, out_hbm, lut_v, r_v, g_v, b_v, or_v, og_v, ob_v):
        in_v = (r_v, g_v, b_v)
        out_v = (or_v, og_v, ob_v)
        wid = lax.axis_index("s") * NC + lax.axis_index("c")
        batch = wid // 4
        base_px = (wid % 4) * per_w
        row0 = 3 * batch

        pltpu.sync_copy(lut_hbm, lut_v)

        def chunk_body(j, _):
            start = base_px + j * CHUNK
            for c in range(3):
                pltpu.sync_copy(x_hbm.at[row0 + c, pl.ds(start, CHUNK)], in_v[c])

            @plsc.parallel_loop(0, CHUNK, L, unroll=2)
            def px_body(off):
                r = in_v[0][pl.ds(off, L)]
                g = in_v[1][pl.ds(off, L)]
                b = in_v[2][pl.ds(off, L)]
                rs = r * inv_binsize
                gs = g * inv_binsize
                bs = b * inv_binsize
                rid = jnp.clip(rs.astype(jnp.int32), 0, DIM - 2)
                gid = jnp.clip(gs.astype(jnp.int32), 0, DIM - 2)
                bid = jnp.clip(bs.astype(jnp.int32), 0, DIM - 2)
                rd = rs - rid.astype(jnp.float32)
                gd = gs - gid.astype(jnp.float32)
                bd = bs - bid.astype(jnp.float32)
                base = rid + gid * DIM + bid * (DIM * DIM)

                ar = 1.0 - rd
                ag = 1.0 - gd
                ab = 1.0 - bd
                p00 = ag * ab
                p10 = gd * ab
                p01 = ag * bd
                p11 = gd * bd
                w = (ar * p00, rd * p00, ar * p10, rd * p10,
                     ar * p01, rd * p01, ar * p11, rd * p11)
                offs = (0, 1, DIM, DIM + 1,
                        DIM * DIM, DIM * DIM + 1, DIM * DIM + DIM, DIM * DIM + DIM + 1)
                for c in range(3):
                    basec = base + c * (DIM ** 3)
                    acc = w[0] * plsc.load_gather(lut_v, [basec])
                    for k in range(1, 8):
                        acc = acc + w[k] * plsc.load_gather(lut_v, [basec + offs[k]])
                    out_v[c][pl.ds(off, L)] = acc

            for c in range(3):
                pltpu.sync_copy(out_v[c], out_hbm.at[row0 + c, pl.ds(start, CHUNK)])
            return 0

        lax.fori_loop(0, n_chunks, chunk_body, 0)

    out = lut_kernel(lut_flat, x_flat)
    return out.reshape(B, C, W, H)
